# baseline (speedup 1.0000x reference)
import math

import jax
import jax.numpy as jnp
from jax.experimental import pallas as pl
from jax.experimental.pallas import tpu as pltpu

_HIDDEN_DIM = 1024
_WORKSPACE_DIM = 256
_MEMORY_TOKEN_DIM = 1024
_TEMPERATURE = 0.25
_RUNGS = [
    (8, 96, 192, 0.55, 0.6, 0.5),
    (6, 128, 256, 0.55, 0.6, 0.5),
    (4, 160, 320, 0.55, 0.6, 0.5),
]
_CAD_PRIOR = 1.0 / (1.0 + math.exp(1.0))

_CHUNK = 256
_GATE_HID = 384


def _gelu(x):
    return jax.nn.gelu(x)


def _ln(x, g, b):
    m = x.mean(-1, keepdims=True)
    v = ((x - m) ** 2).mean(-1, keepdims=True)
    return (x - m) / jnp.sqrt(v + 1e-5) * g + b


def _dot(x, w):
    return jnp.dot(x, w, preferred_element_type=jnp.float32)


def _weight_list(params):
    out = []

    def lin2(p):
        out.append(p["w"])
        out.append(p["b"].reshape(1, -1))

    def mlp2(p):
        lin2(p["l1"])
        lin2(p["l2"])

    mlp2(params["evidence"])
    mlp2(params["ledger_value"])
    lin2(params["ledger_write"])
    lin2(params["ledger_contra"])
    for r in params["rungs"]:
        mlp2(r["key"])
        mlp2(r["value"])
        mlp2(r["refresh"])
        mlp2(r["spawn"])
        mlp2(r["promote"])
        for proj in ("summary_proj", "slot_token_proj"):
            lin2(r[proj]["lin"])
            out.append(r[proj]["ln"]["g"].reshape(1, -1))
            out.append(r[proj]["ln"]["b"].reshape(1, -1))
        mlp2(r["readout"])
    return out


def _copy_plan():
    plan = []
    shapes = _weight_shapes()
    gate_w1_idx = set()
    base = 12
    for (ns, kd, vd, *_t) in _RUNGS:
        for g in range(3):
            gate_w1_idx.add(base + 8 + 4 * g)
        base += 32
    for k, shp in enumerate(shapes):
        if k in gate_w1_idx:
            rung = (k - 12) // 32
            _, kd, vd, *_t = _RUNGS[rung]
            main = _WORKSPACE_DIM + kd + vd
            o_sc = _WORKSPACE_DIM + kd + 2 * vd
            plan.append((k, 0, main, (main, _GATE_HID)))
            plan.append((k, o_sc, 5, (5, _GATE_HID)))
        else:
            plan.append((k, 0, None, shp))
    return plan


def _weight_shapes():
    shapes = [
        (2 * _HIDDEN_DIM, _HIDDEN_DIM), (1, _HIDDEN_DIM),
        (_HIDDEN_DIM, _WORKSPACE_DIM), (1, _WORKSPACE_DIM),
        (256, 512), (1, 512), (512, 256), (1, 256),
        (512, 1), (1, 1), (512, 1), (1, 1),
    ]
    for (ns, kd, vd, *_t) in _RUNGS:
        gd = _WORKSPACE_DIM + kd + 2 * vd + 5
        shapes += [(256, 512), (1, 512), (512, kd), (1, kd)]
        shapes += [(256, 512), (1, 512), (512, vd), (1, vd)]
        for _g in range(3):
            shapes += [(gd, _GATE_HID), (1, _GATE_HID), (_GATE_HID, 1), (1, 1)]
        for _p in range(2):
            shapes += [(vd, _MEMORY_TOKEN_DIM), (1, _MEMORY_TOKEN_DIM),
                       (1, _MEMORY_TOKEN_DIM), (1, _MEMORY_TOKEN_DIM)]
        shapes += [(vd, 512), (1, 512), (512, _MEMORY_TOKEN_DIM),
                   (1, _MEMORY_TOKEN_DIM)]
    return shapes


_PLAN = _copy_plan()
_N_W = len(_weight_shapes())
_N_COPIES = len(_PLAN)


def _body(*args):
    h_ref, m_ref = args[0], args[1]
    wrefs = args[2:2 + _N_W]
    ctx_ref, mt_ref = args[2 + _N_W], args[3 + _N_W]
    acc_ref = args[4 + _N_W]
    vrefs = args[5 + _N_W:5 + _N_W + _N_COPIES]
    sems = args[5 + _N_W + _N_COPIES]

    i = pl.program_id(0)
    nsteps = pl.num_programs(0)

    def copies():
        for c, (k, off, cnt, _shp) in enumerate(_PLAN):
            src = wrefs[k] if cnt is None else wrefs[k].at[pl.ds(off, cnt), :]
            yield pltpu.make_async_copy(src, vrefs[c], sems.at[c])

    @pl.when(i == 0)
    def _start():
        acc_ref[...] = jnp.zeros_like(acc_ref)
        for cp in copies():
            cp.start()

    hb = h_ref[...]
    mb = m_ref[:, pl.ds(i * _CHUNK, _CHUNK)]
    acc_ref[...] += jnp.sum(hb * mb[:, :, None], axis=1)

    @pl.when(i == nsteps - 1)
    def _epilogue():
        for cp in copies():
            cp.wait()

        it = iter(vrefs)

        def nxt():
            return next(it)[...]

        denom = jnp.maximum(jnp.sum(m_ref[...], axis=1, keepdims=True), 1.0)
        pooled = acc_ref[...] / denom
        last = hb[:, -1, :]

        ev_w1, ev_b1, ev_w2, ev_b2 = nxt(), nxt(), nxt(), nxt()
        h1 = _gelu(_dot(pooled, ev_w1[:_HIDDEN_DIM]) +
                   _dot(last, ev_w1[_HIDDEN_DIM:]) + ev_b1)
        ctx = _dot(h1, ev_w2) + ev_b2

        lv_w1, lv_b1, lv_w2, lv_b2 = nxt(), nxt(), nxt(), nxt()
        lv = _dot(_gelu(_dot(ctx, lv_w1) + lv_b1), lv_w2) + lv_b2

        lw_w, lw_b, lc_w, lc_b = nxt(), nxt(), nxt(), nxt()
        wp = jax.nn.sigmoid(_dot(ctx, lw_w[:_WORKSPACE_DIM]) +
                            _dot(lv, lw_w[_WORKSPACE_DIM:]) + lw_b)
        cp_ = jax.nn.sigmoid(_dot(ctx, lc_w[:_WORKSPACE_DIM]) +
                             _dot(lv, lc_w[_WORKSPACE_DIM:]) + lc_b)

        ctx_ref[...] = ctx
        mt_ref[...] = jnp.zeros_like(mt_ref)

        base = 0
        for (ns, kd, vd, rt, st, pt) in _RUNGS:
            k_w1, k_b1, k_w2, k_b2 = nxt(), nxt(), nxt(), nxt()
            v_w1, v_b1, v_w2, v_b2 = nxt(), nxt(), nxt(), nxt()
            ck = _dot(_gelu(_dot(ctx, k_w1) + k_b1), k_w2) + k_b2
            ck = ck / jnp.maximum(
                jnp.sqrt(jnp.sum(ck * ck, axis=-1, keepdims=True)), 1e-6)
            cv = _dot(_gelu(_dot(ctx, v_w1) + v_b1), v_w2) + v_b2

            o_ck = _WORKSPACE_DIM
            o_cv = o_ck + kd
            probs = []
            for _gate in range(3):
                g_main, g_scal = nxt(), nxt()
                g_b1, g_w2, g_b2 = nxt(), nxt(), nxt()
                gh = (_dot(ctx, g_main[:o_ck]) +
                      _dot(ck, g_main[o_ck:o_cv]) +
                      _dot(cv, g_main[o_cv:]) +
                      _CAD_PRIOR * g_scal[1] +
                      g_scal[2] +
                      wp * g_scal[3] +
                      cp_ * g_scal[4] + g_b1)
                probs.append(jax.nn.sigmoid(_dot(_gelu(gh), g_w2) + g_b2))
            rm = jax.nn.sigmoid((probs[0] - rt) / _TEMPERATURE)
            sm = jax.nn.sigmoid((probs[1] - st) / _TEMPERATURE)
            pm = jax.nn.sigmoid((probs[2] - pt) / _TEMPERATURE)

            summary = cv * (rm + sm - rm * sm)
            sp_w, sp_b, sp_g, sp_bb = nxt(), nxt(), nxt(), nxt()
            promoted = pm * _ln(_dot(summary, sp_w) + sp_b, sp_g, sp_bb)
            st_w, st_b, st_g, st_bb = nxt(), nxt(), nxt(), nxt()
            tok0 = _ln(_dot(summary, st_w) + st_b, st_g, st_bb) * jnp.maximum(sm, rm)
            ro_w1, ro_b1, ro_w2, ro_b2 = nxt(), nxt(), nxt(), nxt()
            read = _dot(_gelu(_dot(summary, ro_w1) + ro_b1), ro_w2) + ro_b2

            mt_ref[:, base, :] = tok0
            mt_ref[:, base + ns, :] = read
            mt_ref[:, base + ns + 1, :] = promoted
            base += ns + 2


def kernel(hidden, attention_mask, params):
    B, S, D = hidden.shape
    mask_f = attention_mask.astype(jnp.float32)
    weights = _weight_list(params)

    n_tokens = sum(ns + 2 for (ns, *_rest) in _RUNGS)

    in_specs = [
        pl.BlockSpec((B, _CHUNK, D), lambda i: (0, i, 0)),
        pl.BlockSpec((B, S), lambda i: (0, 0)),
    ]
    in_specs += [pl.BlockSpec(memory_space=pltpu.MemorySpace.HBM)
                 for _ in weights]

    scratch = [pltpu.VMEM((B, D), jnp.float32)]
    scratch += [pltpu.VMEM(shp, jnp.float32) for (_k, _o, _c, shp) in _PLAN]
    scratch += [pltpu.SemaphoreType.DMA((_N_COPIES,))]

    ctx, mt = pl.pallas_call(
        _body,
        grid=(S // _CHUNK,),
        in_specs=in_specs,
        out_specs=[
            pl.BlockSpec((B, _WORKSPACE_DIM), lambda i: (0, 0)),
            pl.BlockSpec((B, n_tokens, _MEMORY_TOKEN_DIM), lambda i: (0, 0, 0)),
        ],
        out_shape=[
            jax.ShapeDtypeStruct((B, _WORKSPACE_DIM), jnp.float32),
            jax.ShapeDtypeStruct((B, n_tokens, _MEMORY_TOKEN_DIM), jnp.float32),
        ],
        scratch_shapes=scratch,
    )(hidden, mask_f, *weights)
    return ctx, mt
